# baseline (speedup 1.0000x reference)
import functools

import jax
import jax.numpy as jnp
from jax import lax
from jax.experimental import pallas as pl
from jax.experimental.pallas import tpu as pltpu
from jax.experimental.pallas import tpu_sc as plsc

NUM_CORES = 2
NUM_SUBCORES = 16
NUM_WORKERS = NUM_CORES * NUM_SUBCORES
LANES = 16
BATCH_N = 16384
FEAT = 64
PAIR = 2 * FEAT
ROWS_PER_W = BATCH_N // NUM_WORKERS
CHUNK = 128
PASS_ROWS = 256
NPASS = ROWS_PER_W // PASS_ROWS
NCHUNK = PASS_ROWS // CHUNK


def _body(user_hbm, item_hbm, upair_hbm, ipair_hbm, out_hbm,
          uidx_v, iidx_v, upidx_v, ipidx_v, urows_v, irows_v, out_v, sem):
    wid = lax.axis_index("s") * NUM_CORES + lax.axis_index("c")
    base = wid * ROWS_PER_W

    pltpu.sync_copy(user_hbm.at[pl.ds(base, ROWS_PER_W)], uidx_v)
    pltpu.sync_copy(item_hbm.at[pl.ds(base, ROWS_PER_W)], iidx_v)

    def pair_body(g, _):
        sl = pl.ds(g * LANES, LANES)
        upidx_v[sl] = uidx_v[sl] >> 1
        ipidx_v[sl] = iidx_v[sl] >> 1
        return ()

    lax.fori_loop(0, ROWS_PER_W // LANES, pair_body, ())

    lane = lax.iota(jnp.int32, LANES)

    def pass_body(p, _):
        lo = p * PASS_ROWS
        copies = []
        for j in range(NCHUNK):
            sl = pl.ds(lo + j * CHUNK, CHUNK)
            dsl = pl.ds(j * CHUNK, CHUNK)
            copies.append(pltpu.async_copy(
                upair_hbm.at[upidx_v.at[sl]], urows_v.at[dsl], sem))
            copies.append(pltpu.async_copy(
                ipair_hbm.at[ipidx_v.at[sl]], irows_v.at[dsl], sem))
        for c in copies:
            c.wait()

        def blk_body(blk, _):
            acc16 = jnp.zeros((LANES,), jnp.float32)
            uvec = uidx_v[pl.ds(lo + blk * LANES, LANES)]
            ivec = iidx_v[pl.ds(lo + blk * LANES, LANES)]
            for rr in range(LANES):
                k = blk * LANES + rr
                ubase = (uvec[rr] & 1) * FEAT
                ibase = (ivec[rr] & 1) * FEAT
                parts = []
                for j in range(FEAT // LANES):
                    u = urows_v[k, pl.ds(ubase + j * LANES, LANES)]
                    i = irows_v[k, pl.ds(ibase + j * LANES, LANES)]
                    parts.append(u * i)
                s = (parts[0] + parts[1]) + (parts[2] + parts[3])
                tot = jnp.sum(s)
                acc16 = jnp.where(lane == rr, tot, acc16)
            out_v[pl.ds(lo + blk * LANES, LANES)] = acc16
            return ()

        lax.fori_loop(0, PASS_ROWS // LANES, blk_body, ())
        return ()

    lax.fori_loop(0, NPASS, pass_body, ())

    pltpu.sync_copy(out_v, out_hbm.at[pl.ds(base, ROWS_PER_W)])


@jax.jit
def kernel(user, item, users_emb, items_emb):
    upair = users_emb.reshape(-1, PAIR)
    ipair = items_emb.reshape(-1, PAIR)
    mesh = plsc.VectorSubcoreMesh(core_axis_name="c", subcore_axis_name="s")
    k = pl.kernel(
        _body,
        out_type=jax.ShapeDtypeStruct((BATCH_N,), jnp.float32),
        mesh=mesh,
        scratch_types=[
            pltpu.VMEM((ROWS_PER_W,), jnp.int32),
            pltpu.VMEM((ROWS_PER_W,), jnp.int32),
            pltpu.VMEM((ROWS_PER_W,), jnp.int32),
            pltpu.VMEM((ROWS_PER_W,), jnp.int32),
            pltpu.VMEM((PASS_ROWS, PAIR), jnp.float32),
            pltpu.VMEM((PASS_ROWS, PAIR), jnp.float32),
            pltpu.VMEM((ROWS_PER_W,), jnp.float32),
            pltpu.SemaphoreType.DMA,
        ],
        compiler_params=pltpu.CompilerParams(needs_layout_passes=False),
    )
    return k(user.astype(jnp.int32), item.astype(jnp.int32), upair, ipair)
